# baseline (speedup 1.0000x reference)
import jax
import jax.numpy as jnp
from jax.experimental import pallas as pl

_B, _H, _W, _C = 4, 112, 112, 384
_OUT = (_B, _H * 2, _W * 2, _C)
_TOTAL = _B * _H * 2 * _W * 2 * _C


def _ident(x_ref, o_ref):
    o_ref[...] = x_ref[...]


def kernel(x, indices):
    flat_idx = indices.reshape(-1)
    flat_vals = x.reshape(-1)
    unpooled = jnp.zeros((_TOTAL,), dtype=x.dtype).at[flat_idx].add(flat_vals)
    out = unpooled.reshape(_OUT)
    probe = pl.pallas_call(
        _ident, out_shape=jax.ShapeDtypeStruct((8, 128), jnp.float32)
    )(out[0, :8, 0, :128])
    return out.at[0, :8, 0, :128].set(probe)


# speedup vs baseline: 1.2707x; 1.2707x over previous
import functools

import jax
import jax.numpy as jnp
from jax import lax
from jax.experimental import pallas as pl
from jax.experimental.pallas import tpu as pltpu
from jax.experimental.pallas import tpu_sc as plsc

_B, _H, _W, _C = 4, 112, 112, 384
N = _B * _H * _W * _C
TOTAL = N * 4
OUT_SHAPE = (_B, _H * 2, _W * 2, _C)

NC, NS = 2, 16
NW = NC * NS
NP_W = N // NW
CHUNK = 2048
GPC = CHUNK // 16
NCHUNK_W = NP_W // CHUNK

WIN = 7 * (1 << 18)
NWIN = TOTAL // WIN
NWIN_SC = NWIN // 2
SLICE = WIN // NS
ZCH = 8192

HPAD = 48
TSIZE = N + NWIN * (NW * 16 + CHUNK)
SENT = 1 << 30
SPOS = 1 << 30

_mesh = plsc.VectorSubcoreMesh(core_axis_name="c", subcore_axis_name="s")
_cparams = pltpu.CompilerParams(needs_layout_passes=False)

_i32 = jnp.int32


def _iota16():
    return lax.iota(_i32, 16)


def _bucket_of(v):
    y = lax.shift_right_logical(v, 18)
    return lax.shift_right_logical(y * 9363, 16)


def _extract(ref, p):
    base = pl.multiple_of((lax.shift_right_logical(p, 4)) * 16, 16)
    v = ref[pl.ds(base, 16)]
    lane = lax.bitwise_and(p, 15)
    return jnp.sum(jnp.where(_iota16() == lane, v, 0))


def _roundup16(v):
    return lax.bitwise_and(v + 15, ~15)


@functools.partial(
    pl.kernel,
    out_type=jax.ShapeDtypeStruct((NW * HPAD,), _i32),
    mesh=_mesh,
    compiler_params=_cparams,
    scratch_types=[
        pltpu.VMEM((CHUNK,), _i32),
        pltpu.VMEM((HPAD,), _i32),
    ],
)
def _k1_hist(idx_hbm, hist_hbm, buf, hist):
    wid = lax.axis_index("c") * NS + lax.axis_index("s")
    for j in range(HPAD // 16):
        hist[pl.ds(j * 16, 16)] = jnp.zeros((16,), _i32)

    @pl.loop(0, NCHUNK_W)
    def _chunks(ch):
        off = pl.multiple_of(wid * NP_W + ch * CHUNK, CHUNK)
        pltpu.sync_copy(idx_hbm.at[pl.ds(off, CHUNK)], buf)

        @pl.loop(0, GPC)
        def _groups(g):
            v = buf[pl.ds(g * 16, 16)]
            b = _bucket_of(v)
            rank, last = plsc.scan_count(b)
            plsc.addupdate_scatter(hist, [b], rank, mask=last)

    pltpu.sync_copy(hist, hist_hbm.at[pl.ds(wid * HPAD, HPAD)])


@functools.partial(
    pl.kernel,
    out_type=(
        jax.ShapeDtypeStruct((TSIZE,), _i32),
        jax.ShapeDtypeStruct((TSIZE,), jnp.float32),
        jax.ShapeDtypeStruct((2 * HPAD,), _i32),
    ),
    mesh=_mesh,
    compiler_params=_cparams,
    scratch_types=[
        pltpu.VMEM((NW * HPAD,), _i32),
        pltpu.VMEM((HPAD,), _i32),
        pltpu.VMEM((HPAD,), _i32),
        pltpu.VMEM((2 * HPAD,), _i32),
        pltpu.VMEM((CHUNK,), _i32),
        pltpu.VMEM((CHUNK,), jnp.float32),
        pltpu.VMEM((CHUNK,), _i32),
        pltpu.VMEM((CHUNK,), _i32),
        pltpu.VMEM((16,), _i32),
    ],
)
def _k2_part(idx_hbm, val_hbm, hist_hbm, t_idx, t_val, meta_hbm,
             hall, cursor, endr, meta, idx_v, val_v, pos_v, lid_v, sbuf):
    wid = lax.axis_index("c") * NS + lax.axis_index("s")
    pltpu.sync_copy(hist_hbm, hall)
    sbuf[...] = jnp.full((16,), SENT, _i32)

    nv = HPAD // 16
    cntp = [jnp.zeros((16,), _i32) for _ in range(nv)]
    for w in range(NW):
        for j in range(nv):
            cntp[j] = cntp[j] + _roundup16(hall[pl.ds(w * HPAD + j * 16, 16)])
    carry = jnp.zeros((), _i32)
    for j in range(nv):
        cap = lax.bitwise_and(cntp[j] + (CHUNK - 1), ~(CHUNK - 1))
        cs = plsc.cumsum(cap)
        base = cs - cap + carry
        carry = carry + jnp.sum(cap)
        meta[pl.ds(j * 16, 16)] = base
        meta[pl.ds(HPAD + j * 16, 16)] = cntp[j]
        cursor[pl.ds(j * 16, 16)] = base

    @pl.loop(0, NW)
    def _prior(w):
        @pl.when(w < wid)
        def _():
            for j in range(nv):
                cursor[pl.ds(j * 16, 16)] = cursor[pl.ds(j * 16, 16)] + (
                    _roundup16(hall[pl.ds(w * HPAD + j * 16, 16)]))

    own = wid * HPAD
    for j in range(nv):
        endr[pl.ds(j * 16, 16)] = cursor[pl.ds(j * 16, 16)] + _roundup16(
            hall[pl.ds(own + j * 16, 16)])

    @pl.when(wid == 0)
    def _():
        pltpu.sync_copy(meta, meta_hbm)

    @pl.loop(0, NCHUNK_W)
    def _chunks(ch):
        off = pl.multiple_of(wid * NP_W + ch * CHUNK, CHUNK)
        pltpu.sync_copy(idx_hbm.at[pl.ds(off, CHUNK)], idx_v)
        pltpu.sync_copy(val_hbm.at[pl.ds(off, CHUNK)], val_v)

        @pl.loop(0, GPC)
        def _groups(g):
            v = idx_v[pl.ds(g * 16, 16)]
            b = _bucket_of(v)
            rank, last = plsc.scan_count(b)
            cur = plsc.load_gather(cursor, [b])
            pos_v[pl.ds(g * 16, 16)] = cur + rank - 1
            lid_v[pl.ds(g * 16, 16)] = v - b * WIN
            plsc.store_scatter(cursor, [b], cur + rank, mask=last)

        pltpu.sync_copy(lid_v, t_idx.at[pos_v])
        pltpu.sync_copy(val_v, t_val.at[pos_v])

    @pl.loop(0, NWIN)
    def _fill(b):
        p0 = _extract(cursor, b)
        pe = _extract(endr, b)
        pos = p0 + _iota16()
        pos = jnp.where(pos < pe, pos, SPOS)
        pltpu.sync_copy(
            sbuf, t_idx.at[plsc.Indices(pos, ignored_value=SPOS)])


@functools.partial(
    pl.kernel,
    out_type=jax.ShapeDtypeStruct((TOTAL,), jnp.float32),
    mesh=_mesh,
    compiler_params=_cparams,
    scratch_types=[
        pltpu.VMEM_SHARED((WIN,), jnp.float32),
        pltpu.VMEM((2 * HPAD,), _i32),
        pltpu.VMEM((CHUNK,), _i32),
        pltpu.VMEM((CHUNK,), jnp.float32),
        pltpu.VMEM((ZCH,), jnp.float32),
    ],
)
def _k3_acc(t_idx, t_val, meta_hbm, out_hbm, window, meta, lid_v, val_v, zbuf):
    sc = lax.axis_index("c")
    tid = lax.axis_index("s")
    pltpu.sync_copy(meta_hbm, meta)

    @pl.loop(0, ZCH // 16)
    def _z(g):
        zbuf[pl.ds(g * 16, 16)] = jnp.zeros((16,), jnp.float32)

    @pl.loop(0, NWIN_SC)
    def _windows(k):
        w = 2 * k + sc
        base_w = pl.multiple_of(_extract(meta, w), CHUNK)
        cnt_w = _extract(meta, HPAD + w)

        for j in range(SLICE // ZCH):
            dst = pl.multiple_of(tid * SLICE + j * ZCH, ZCH)
            pltpu.sync_copy(zbuf, window.at[pl.ds(dst, ZCH)])
        plsc.subcore_barrier()

        trips = lax.shift_right_logical(cnt_w + (CHUNK - 1), 11)

        @pl.loop(tid, trips, step=NS)
        def _chunk(c):
            off = pl.multiple_of(base_w + c * CHUNK, CHUNK)
            pltpu.sync_copy(t_idx.at[pl.ds(off, CHUNK)], lid_v)
            pltpu.sync_copy(t_val.at[pl.ds(off, CHUNK)], val_v)

            @pl.when(c == trips - 1)
            def _tail():
                rem = cnt_w - c * CHUNK

                @pl.loop(lax.shift_right_logical(rem, 4), GPC)
                def _mask(g):
                    s = pl.multiple_of(g * 16, 16)
                    vv = lid_v[pl.ds(s, 16)]
                    keep = (s + _iota16()) < rem
                    lid_v[pl.ds(s, 16)] = jnp.where(keep, vv, SENT)

            pltpu.sync_copy(
                val_v,
                window.at[plsc.Indices(lid_v, ignored_value=SENT)],
                add=True,
            )

        plsc.subcore_barrier()
        src = pl.multiple_of(tid * SLICE, ZCH)
        dst = pl.multiple_of(w * WIN + tid * SLICE, ZCH)
        pltpu.sync_copy(window.at[pl.ds(src, SLICE)],
                        out_hbm.at[pl.ds(dst, SLICE)])


def kernel(x, indices):
    idx = indices.reshape(-1).astype(_i32)
    val = x.reshape(-1)
    hist = _k1_hist(idx)
    t_idx, t_val, meta = _k2_part(idx, val, hist)
    out = _k3_acc(t_idx, t_val, meta)
    return out.reshape(OUT_SHAPE)


# speedup vs baseline: 12.9712x; 10.2082x over previous
import functools

import jax
import jax.numpy as jnp
from jax import lax
from jax.experimental import pallas as pl
from jax.experimental.pallas import tpu as pltpu
from jax.experimental.pallas import tpu_sc as plsc

_B, _H, _W, _C = 4, 112, 112, 384
N = _B * _H * _W * _C
TOTAL = N * 4
OUT_SHAPE = (_B, _H * 2, _W * 2, _C)

NC, NS = 2, 16
NW = NC * NS
NP_W = N // NW

M = 28672
KB = NP_W // M
NR = NW * KB
GPB = M // 16

WIN = 7 * (1 << 18)
NWIN = TOTAL // WIN
NWIN_SC = NWIN // 2
SLICE = WIN // NS
ZCH = 8192

CH3 = 1024
NG3 = CH3 // 16

WPR = 32
NRT = NW * WPR
STS = 48 * NRT
TSIZE = N + CH3
SENT = 1 << 30
SPOS = 1 << 30

_mesh = plsc.VectorSubcoreMesh(core_axis_name="c", subcore_axis_name="s")
_cparams = pltpu.CompilerParams(needs_layout_passes=False)

_i32 = jnp.int32


def _iota16():
    return lax.iota(_i32, 16)


def _bucket_of(v):
    y = lax.shift_right_logical(v, 18)
    return lax.shift_right_logical(y * 9363, 16)


def _extract(ref, p):
    base = pl.multiple_of(lax.shift_right_logical(p, 4) * 16, 16)
    v = ref[pl.ds(base, 16)]
    lane = lax.bitwise_and(p, 15)
    return jnp.sum(jnp.where(_iota16() == lane, v, 0))


@functools.partial(
    pl.kernel,
    out_type=(
        jax.ShapeDtypeStruct((TSIZE,), _i32),
        jax.ShapeDtypeStruct((TSIZE,), jnp.float32),
        jax.ShapeDtypeStruct((STS,), _i32),
    ),
    mesh=_mesh,
    compiler_params=_cparams,
    scratch_types=[
        pltpu.VMEM((M,), _i32),
        pltpu.VMEM((M,), jnp.float32),
        pltpu.VMEM((M,), _i32),
        pltpu.VMEM((M,), jnp.float32),
        pltpu.VMEM((48,), _i32),
        pltpu.VMEM((48,), _i32),
        pltpu.VMEM((16,), _i32),
    ],
)
def _k1_part(idx_hbm, val_hbm, t_idx, t_val, starts_hbm,
             idx_v, val_v, lid_s, val_s, cnt, cursor, svec):
    wid = lax.axis_index("c") * NS + lax.axis_index("s")

    @pl.loop(0, KB)
    def _blocks(k):
        r = wid * KB + k
        base_r = pl.multiple_of(r * M, M)
        pltpu.sync_copy(idx_hbm.at[pl.ds(base_r, M)], idx_v)
        pltpu.sync_copy(val_hbm.at[pl.ds(base_r, M)], val_v)

        for j in range(3):
            cnt[pl.ds(j * 16, 16)] = jnp.zeros((16,), _i32)

        @pl.loop(0, GPB, unroll=2)
        def _hist(g):
            b = _bucket_of(idx_v[pl.ds(g * 16, 16)])
            rank, last = plsc.scan_count(b)
            plsc.addupdate_scatter(cnt, [b], rank, mask=last)

        carry = jnp.zeros((), _i32)
        for j in range(3):
            c = cnt[pl.ds(j * 16, 16)]
            excl = plsc.cumsum(c) - c + carry
            carry = carry + jnp.sum(c)
            cursor[pl.ds(j * 16, 16)] = excl
            svec[...] = excl + base_r
            blane = j * 16 + _iota16()
            rp = wid * WPR + k
            pos = jnp.where(blane <= NWIN, blane * NRT + rp, SPOS)
            pltpu.sync_copy(
                svec, starts_hbm.at[plsc.Indices(pos, ignored_value=SPOS)])

        @pl.loop(0, GPB, unroll=2)
        def _scatter(g):
            v = idx_v[pl.ds(g * 16, 16)]
            b = _bucket_of(v)
            rank, last = plsc.scan_count(b)
            cur = plsc.load_gather(cursor, [b])
            pos = cur + rank - 1
            plsc.store_scatter(lid_s, [pos], v - b * WIN)
            plsc.store_scatter(val_s, [pos], val_v[pl.ds(g * 16, 16)])
            plsc.store_scatter(cursor, [b], cur + rank, mask=last)

        pltpu.sync_copy(lid_s, t_idx.at[pl.ds(base_r, M)])
        pltpu.sync_copy(val_s, t_val.at[pl.ds(base_r, M)])


@functools.partial(
    pl.kernel,
    out_type=jax.ShapeDtypeStruct((TOTAL,), jnp.float32),
    mesh=_mesh,
    compiler_params=_cparams,
    scratch_types=[
        pltpu.VMEM_SHARED((WIN,), jnp.float32),
        pltpu.VMEM((NRT,), _i32),
        pltpu.VMEM((NRT,), _i32),
        pltpu.VMEM((CH3,), _i32),
        pltpu.VMEM((CH3,), jnp.float32),
        pltpu.VMEM((ZCH,), jnp.float32),
    ],
)
def _k2_acc(t_idx, t_val, starts_hbm, out_hbm,
            window, s_w, s_w1, lid_v, val_v, zbuf):
    sc = lax.axis_index("c")
    tid = lax.axis_index("s")

    @pl.loop(0, ZCH // 16)
    def _z(g):
        zbuf[pl.ds(g * 16, 16)] = jnp.zeros((16,), jnp.float32)

    @pl.loop(0, NWIN_SC)
    def _windows(k):
        w = 2 * k + sc
        pltpu.sync_copy(
            starts_hbm.at[pl.ds(pl.multiple_of(w * NRT, 16), NRT)], s_w)
        pltpu.sync_copy(
            starts_hbm.at[pl.ds(pl.multiple_of((w + 1) * NRT, 16), NRT)],
            s_w1)

        for j in range(SLICE // ZCH):
            dst = pl.multiple_of(tid * SLICE + j * ZCH, ZCH)
            pltpu.sync_copy(zbuf, window.at[pl.ds(dst, ZCH)])
        plsc.subcore_barrier()

        def _do_run(r):
            s = _extract(s_w, r)
            e = _extract(s_w1, r)
            s0 = pl.multiple_of(lax.bitwise_and(s, ~15), 16)
            nj = lax.shift_right_logical(e - s0 + (CH3 - 1), 10)

            @pl.loop(0, nj)
            def _chunk(j):
                cj = pl.multiple_of(s0 + j * CH3, 16)
                pltpu.sync_copy(t_idx.at[pl.ds(cj, CH3)], lid_v)
                pltpu.sync_copy(t_val.at[pl.ds(cj, CH3)], val_v)
                a = jnp.maximum(s - cj, 0)
                bb = jnp.minimum(e - cj, CH3)

                def _mask(g):
                    gs = pl.multiple_of(g * 16, 16)
                    p = gs + _iota16()
                    keep = (p >= a) & (p < bb)
                    lid_v[pl.ds(gs, 16)] = jnp.where(
                        keep, lid_v[pl.ds(gs, 16)], SENT)

                pl.loop(0, lax.shift_right_logical(a + 15, 4))(_mask)
                pl.loop(lax.shift_right_logical(bb, 4), NG3)(_mask)

                pltpu.sync_copy(
                    val_v,
                    window.at[plsc.Indices(lid_v, ignored_value=SENT)],
                    add=True,
                )

        @pl.loop(tid, NRT, step=NS)
        def _runs(r):
            @pl.when(lax.bitwise_and(r, WPR - 1) < KB)
            def _valid():
                _do_run(r)

        plsc.subcore_barrier()
        src = pl.multiple_of(tid * SLICE, ZCH)
        dst = pl.multiple_of(w * WIN + tid * SLICE, ZCH)
        pltpu.sync_copy(window.at[pl.ds(src, SLICE)],
                        out_hbm.at[pl.ds(dst, SLICE)])


def kernel(x, indices):
    idx = indices.reshape(-1).astype(_i32)
    val = x.reshape(-1)
    t_idx, t_val, starts = _k1_part(idx, val)
    out = _k2_acc(t_idx, t_val, starts)
    return out.reshape(OUT_SHAPE)


# speedup vs baseline: 27.6458x; 2.1313x over previous
import functools

import jax
import jax.numpy as jnp
from jax import lax
from jax.experimental import pallas as pl
from jax.experimental.pallas import tpu as pltpu
from jax.experimental.pallas import tpu_sc as plsc

_B, _H, _W, _C = 4, 112, 112, 384
N = _B * _H * _W * _C
TOTAL = N * 4
OUT_SHAPE = (_B, _H * 2, _W * 2, _C)

NC, NS = 2, 16
NW = NC * NS
NP_W = N // NW

M = 28672
KB = NP_W // M
NR = NW * KB
GPB = M // 16

WIN = 7 * (1 << 18)
NWIN = TOTAL // WIN
NWIN_SC = NWIN // 2
SLICE = WIN // NS
ZCH = 4096

CH3 = 1024
NG3 = CH3 // 16
NB = 4

WPR = 32
NRT = NW * WPR
SPT = NRT // NS
STS = 48 * NRT
TSIZE = N + CH3
SENT = 1 << 30
SPOS = 1 << 30

_mesh = plsc.VectorSubcoreMesh(core_axis_name="c", subcore_axis_name="s")
_cparams = pltpu.CompilerParams(needs_layout_passes=False)

_i32 = jnp.int32


def _iota16():
    return lax.iota(_i32, 16)


def _bucket_of(v):
    y = lax.shift_right_logical(v, 18)
    return lax.shift_right_logical(y * 9363, 16)


def _extract(ref, p):
    base = pl.multiple_of(lax.shift_right_logical(p, 4) * 16, 16)
    v = ref[pl.ds(base, 16)]
    lane = lax.bitwise_and(p, 15)
    return jnp.sum(jnp.where(_iota16() == lane, v, 0))


@functools.partial(
    pl.kernel,
    out_type=(
        jax.ShapeDtypeStruct((TSIZE,), _i32),
        jax.ShapeDtypeStruct((TSIZE,), jnp.float32),
        jax.ShapeDtypeStruct((STS,), _i32),
    ),
    mesh=_mesh,
    compiler_params=_cparams,
    scratch_types=[
        pltpu.VMEM((M,), _i32),
        pltpu.VMEM((M,), jnp.float32),
        pltpu.VMEM((M,), _i32),
        pltpu.VMEM((M,), jnp.float32),
        pltpu.VMEM((1536,), _i32),
        pltpu.VMEM((1536,), _i32),
        pltpu.VMEM((16,), _i32),
        pltpu.SemaphoreType.DMA,
        pltpu.SemaphoreType.DMA,
    ],
)
def _k1_part(idx_hbm, val_hbm, t_idx, t_val, starts_hbm,
             idx_v, val_v, lid_s, val_s, cnt2, cursor2, svec, sw, sf):
    wid = lax.axis_index("c") * NS + lax.axis_index("s")
    ones = jnp.ones((16,), _i32)

    @pl.loop(0, KB)
    def _blocks(k):
        r = wid * KB + k
        base_r = pl.multiple_of(r * M, M)
        di = pltpu.make_async_copy(idx_hbm.at[pl.ds(base_r, M)], idx_v, sf)
        dv = pltpu.make_async_copy(val_hbm.at[pl.ds(base_r, M)], val_v, sf)
        di.start()
        dv.start()
        di.wait()
        dv.wait()

        @pl.loop(0, 96)
        def _zero(b):
            cnt2[pl.ds(pl.multiple_of(b * 16, 16), 16)] = jnp.zeros(
                (16,), _i32)

        @pl.loop(0, GPB // 2, unroll=4)
        def _hist(g):
            bA = _bucket_of(idx_v[pl.ds(g * 16, 16)])
            bB = _bucket_of(idx_v[pl.ds((GPB // 2) * 16 + g * 16, 16)])
            addrA = lax.bitwise_or(lax.shift_left(bA, 5), _iota16())
            addrB = lax.bitwise_or(lax.shift_left(bB, 5), 16 + _iota16())
            plsc.addupdate_scatter(cnt2, [addrA], ones)
            plsc.addupdate_scatter(cnt2, [addrB], ones)

        carry = jnp.zeros((), _i32)
        for b in range(NWIN):
            cA = cnt2[pl.ds(b * 32, 16)]
            cursor2[pl.ds(b * 32, 16)] = plsc.cumsum(cA) - cA + carry
            carry = carry + jnp.sum(cA)
            cB = cnt2[pl.ds(b * 32 + 16, 16)]
            cursor2[pl.ds(b * 32 + 16, 16)] = plsc.cumsum(cB) - cB + carry
            carry = carry + jnp.sum(cB)
        cursor2[pl.ds(NWIN * 32, 16)] = jnp.broadcast_to(carry, (16,))

        rp = wid * WPR + k
        for j in range(3):
            blane = j * 16 + _iota16()
            svec[...] = plsc.load_gather(
                cursor2, [lax.shift_left(blane, 5)]) + base_r
            pos = jnp.where(blane <= NWIN, blane * NRT + rp, SPOS)
            pltpu.sync_copy(
                svec, starts_hbm.at[plsc.Indices(pos, ignored_value=SPOS)])

        @pl.when(k > 0)
        def _drain_prev():
            pltpu.make_async_copy(lid_s, t_idx.at[pl.ds(0, M)], sw).wait()
            pltpu.make_async_copy(val_s, t_val.at[pl.ds(0, M)], sw).wait()

        @pl.loop(0, GPB // 2, unroll=4)
        def _scatter(g):
            gB = (GPB // 2) * 16 + g * 16
            vA = idx_v[pl.ds(g * 16, 16)]
            vB = idx_v[pl.ds(gB, 16)]
            bA = _bucket_of(vA)
            bB = _bucket_of(vB)
            addrA = lax.bitwise_or(lax.shift_left(bA, 5), _iota16())
            addrB = lax.bitwise_or(lax.shift_left(bB, 5), 16 + _iota16())
            curA = plsc.load_gather(cursor2, [addrA])
            curB = plsc.load_gather(cursor2, [addrB])
            plsc.store_scatter(lid_s, [curA], vA - bA * WIN)
            plsc.store_scatter(lid_s, [curB], vB - bB * WIN)
            plsc.store_scatter(val_s, [curA], val_v[pl.ds(g * 16, 16)])
            plsc.store_scatter(val_s, [curB], val_v[pl.ds(gB, 16)])
            plsc.store_scatter(cursor2, [addrA], curA + 1)
            plsc.store_scatter(cursor2, [addrB], curB + 1)

        pltpu.make_async_copy(lid_s, t_idx.at[pl.ds(base_r, M)], sw).start()
        pltpu.make_async_copy(val_s, t_val.at[pl.ds(base_r, M)], sw).start()

    pltpu.make_async_copy(lid_s, t_idx.at[pl.ds(0, M)], sw).wait()
    pltpu.make_async_copy(val_s, t_val.at[pl.ds(0, M)], sw).wait()


@functools.partial(
    pl.kernel,
    out_type=jax.ShapeDtypeStruct((TOTAL,), jnp.float32),
    mesh=_mesh,
    compiler_params=_cparams,
    scratch_types=[
        pltpu.VMEM_SHARED((WIN,), jnp.float32),
        pltpu.VMEM((NRT,), _i32),
        pltpu.VMEM((NRT,), _i32),
        pltpu.VMEM((NB * CH3,), _i32),
        pltpu.VMEM((NB * CH3,), jnp.float32),
        pltpu.VMEM((ZCH,), jnp.float32),
        pltpu.SemaphoreType.DMA,
        pltpu.SemaphoreType.DMA,
    ],
)
def _k2_acc(t_idx, t_val, starts_hbm, out_hbm,
            window, s_w, s_w1, lid_b, val_b, zbuf, sb, sz):
    sc = lax.axis_index("c")
    tid = lax.axis_index("s")

    @pl.loop(0, ZCH // 16)
    def _z(g):
        zbuf[pl.ds(g * 16, 16)] = jnp.zeros((16,), jnp.float32)

    @pl.loop(0, NWIN_SC)
    def _windows(k):
        w = 2 * k + sc
        pltpu.sync_copy(
            starts_hbm.at[pl.ds(pl.multiple_of(w * NRT, 16), NRT)], s_w)
        pltpu.sync_copy(
            starts_hbm.at[pl.ds(pl.multiple_of((w + 1) * NRT, 16), NRT)],
            s_w1)

        zdescs = []
        for i in range(SLICE // ZCH):
            dst = pl.multiple_of(tid * SLICE + i * ZCH, ZCH)
            zdescs.append(
                pltpu.make_async_copy(zbuf, window.at[pl.ds(dst, ZCH)], sz))
        for zd in zdescs:
            zd.start()
        for zd in zdescs:
            zd.wait()
        plsc.subcore_barrier()

        @pl.loop(0, SPT // NB)
        def _batch(bi):
            q0 = bi * NB
            ss, ee, s0s, njs = [], [], [], []
            for d in range(NB):
                q = q0 + d
                rr = tid * SPT + q
                valid = lax.bitwise_and(q, WPR - 1) < KB
                s = jnp.where(valid, _extract(s_w, rr), 0)
                e = jnp.where(valid, _extract(s_w1, rr), 0)
                s0 = lax.bitwise_and(s, ~15)
                nj = jnp.where(
                    valid,
                    lax.shift_right_logical(e - s0 + (CH3 - 1), 10), 0)
                ss.append(s)
                ee.append(e)
                s0s.append(s0)
                njs.append(nj)
            njmax = njs[0]
            for d in range(1, NB):
                njmax = jnp.maximum(njmax, njs[d])

            @pl.loop(0, njmax)
            def _round(j):
                cjs, descs = [], []
                for d in range(NB):
                    cj = pl.multiple_of(s0s[d] + j * CH3, 16)
                    cjs.append(cj)
                    descs.append((
                        pltpu.make_async_copy(
                            t_idx.at[pl.ds(cj, CH3)],
                            lid_b.at[pl.ds(d * CH3, CH3)], sb),
                        pltpu.make_async_copy(
                            t_val.at[pl.ds(cj, CH3)],
                            val_b.at[pl.ds(d * CH3, CH3)], sb),
                    ))
                for d in range(NB):
                    @pl.when(j < njs[d])
                    def _fire(d=d):
                        descs[d][0].start()
                        descs[d][1].start()
                for d in range(NB):
                    @pl.when(j < njs[d])
                    def _drain(d=d):
                        descs[d][0].wait()
                        descs[d][1].wait()

                sent16 = jnp.full((16,), SENT, _i32)
                for d in range(NB):
                    a = jnp.maximum(ss[d] - cjs[d], 0)
                    bb = jnp.minimum(ee[d] - cjs[d], CH3)

                    def _mask(g, d=d, a=a, bb=bb):
                        gs = pl.multiple_of(d * CH3 + g * 16, 16)
                        p = g * 16 + _iota16()
                        keep = (p >= a) & (p < bb)
                        lid_b[pl.ds(gs, 16)] = jnp.where(
                            keep, lid_b[pl.ds(gs, 16)], SENT)

                    @pl.when(j < njs[d])
                    def _edges(d=d, a=a, bb=bb, _mask=_mask):
                        pl.loop(0, lax.shift_right_logical(a + 15, 4))(_mask)
                        pl.loop(lax.shift_right_logical(bb, 4), NG3)(_mask)

                    @pl.when(j >= njs[d])
                    def _fill(d=d):
                        @pl.loop(0, NG3, unroll=4)
                        def _f(g, d=d):
                            gs = pl.multiple_of(d * CH3 + g * 16, 16)
                            lid_b[pl.ds(gs, 16)] = sent16

                pltpu.sync_copy(
                    val_b,
                    window.at[plsc.Indices(lid_b, ignored_value=SENT)],
                    add=True,
                )

        plsc.subcore_barrier()
        src = pl.multiple_of(tid * SLICE, ZCH)
        dst = pl.multiple_of(w * WIN + tid * SLICE, ZCH)
        pltpu.sync_copy(window.at[pl.ds(src, SLICE)],
                        out_hbm.at[pl.ds(dst, SLICE)])


def kernel(x, indices):
    idx = indices.reshape(-1).astype(_i32)
    val = x.reshape(-1)
    t_idx, t_val, starts = _k1_part(idx, val)
    out = _k2_acc(t_idx, t_val, starts)
    return out.reshape(OUT_SHAPE)
